# baseline (speedup 1.0000x reference)
import functools

import jax
import jax.numpy as jnp
from jax import lax
from jax.experimental import pallas as pl
from jax.experimental.pallas import tpu as pltpu
from jax.experimental.pallas import tpu_sc as plsc

N = 10000
D_IN = 128
D_HID = 64
D_OUT2 = 8
E = 320000

NC = 2
NS = 16
CH = 128
CPT_SUM = 157
NCHT = NS * CPT_SUM
E_PAD = NCHT * CH
NPAD = 10240
ROWS_PT = NPAD // NS

SPLIT_DEG = (79, 78)
SPLIT1 = (110, 47)
SPLIT2 = (88, 69)

BN = 1000
NB = N // BN

_MESH = plsc.VectorSubcoreMesh(core_axis_name="c", subcore_axis_name="s")
_SC_PARAMS = pltpu.CompilerParams(use_tc_tiling_on_sc=False)


def _sc_degree(dst2, ones_h, zeros_h):
    cpt_max = max(SPLIT_DEG)

    @functools.partial(
        pl.kernel,
        out_type=jax.ShapeDtypeStruct((NC, NPAD, 1), jnp.float32),
        mesh=_MESH,
        scratch_types=[
            pltpu.VMEM((cpt_max, CH), jnp.int32),
            pltpu.VMEM((CH, 1), jnp.float32),
            pltpu.VMEM_SHARED((NPAD, 1), jnp.float32),
        ],
        compiler_params=_SC_PARAMS,
    )
    def k(dst_h, ones_hbm, zeros_hbm, parts, didx, ones_v, dacc):
        c = lax.axis_index("c")
        s = lax.axis_index("s")
        pltpu.sync_copy(zeros_hbm, dacc.at[pl.ds(s * ROWS_PT, ROWS_PT)])
        pltpu.sync_copy(ones_hbm, ones_v)

        for ci in range(NC):
            cpt = SPLIT_DEG[ci]
            base = 0 if ci == 0 else NS * SPLIT_DEG[0]

            @pl.when(c == ci)
            def _(cpt=cpt, base=base):
                start = base + s * cpt
                pltpu.sync_copy(dst_h.at[pl.ds(start, cpt)],
                                didx.at[pl.ds(0, cpt)])
                plsc.subcore_barrier()

                @pl.loop(0, cpt)
                def _(j):
                    pltpu.sync_copy(ones_v, dacc.at[didx.at[j]], add=True)

        plsc.subcore_barrier()
        pltpu.sync_copy(
            dacc.at[pl.ds(s * ROWS_PT, ROWS_PT)],
            parts.at[c, pl.ds(s * ROWS_PT, ROWS_PT)],
        )

    return k(dst2, ones_h, zeros_h)


def _sc_aggregate(table, src2, dst2, zeros_h, width, split):
    cpt_max = max(split)

    @functools.partial(
        pl.kernel,
        out_type=jax.ShapeDtypeStruct((NC, NPAD, width), jnp.float32),
        mesh=_MESH,
        scratch_types=[
            pltpu.VMEM((cpt_max, CH), jnp.int32),
            pltpu.VMEM((cpt_max, CH), jnp.int32),
            [pltpu.VMEM((CH, width), jnp.float32) for _ in range(4)],
            pltpu.VMEM_SHARED((NPAD, width), jnp.float32),
            [pltpu.SemaphoreType.DMA for _ in range(4)],
        ],
        compiler_params=_SC_PARAMS,
    )
    def k(tab_h, src_h, dst_h, zeros_hbm, parts,
          sidx, didx, rows, acc, sems):
        c = lax.axis_index("c")
        s = lax.axis_index("s")

        @pl.loop(0, ROWS_PT // CH)
        def _(kk):
            pltpu.sync_copy(zeros_hbm, acc.at[pl.ds(s * ROWS_PT + kk * CH, CH)])

        for ci in range(NC):
            cpt = split[ci]
            base = 0 if ci == 0 else NS * split[0]

            @pl.when(c == ci)
            def _(cpt=cpt, base=base):
                start = base + s * cpt
                pltpu.sync_copy(src_h.at[pl.ds(start, cpt)],
                                sidx.at[pl.ds(0, cpt)])
                pltpu.sync_copy(dst_h.at[pl.ds(start, cpt)],
                                didx.at[pl.ds(0, cpt)])
                plsc.subcore_barrier()

                for j in range(3):
                    pltpu.make_async_copy(
                        tab_h.at[sidx.at[j]], rows[j], sems[j]).start()

                @pl.loop(0, (cpt + 3) // 4)
                def _(i):
                    for b in range(4):
                        j = 4 * i + b
                        nxt = j + 3
                        bn = (b + 3) % 4

                        @pl.when(nxt < cpt)
                        def _():
                            pltpu.make_async_copy(
                                tab_h.at[sidx.at[nxt]], rows[bn], sems[bn]
                            ).start()

                        @pl.when(j < cpt)
                        def _():
                            pltpu.make_async_copy(
                                tab_h.at[sidx.at[j]], rows[b], sems[b]
                            ).wait()
                            pltpu.sync_copy(rows[b], acc.at[didx.at[j]],
                                            add=True)

        plsc.subcore_barrier()

        @pl.loop(0, ROWS_PT // CH)
        def _(kk):
            r = s * ROWS_PT + kk * CH
            pltpu.sync_copy(acc.at[pl.ds(r, CH)], parts.at[c, pl.ds(r, CH)])

    return k(table, src2, dst2, zeros_h)


def _tc_matmul1(x, W1):
    def body(x_ref, w_ref, o_ref):
        o_ref[...] = jnp.dot(x_ref[...], w_ref[...],
                             preferred_element_type=jnp.float32)

    return pl.pallas_call(
        body, out_shape=jax.ShapeDtypeStruct((N, D_HID), jnp.float32)
    )(x, W1)


def _tc_scale(h1, deg_parts):
    def body(h_ref, dp_ref, g_ref, dinv_ref):
        deg = dp_ref[0, :N] + dp_ref[1, :N] + 1.0
        dinv = lax.rsqrt(deg)
        dinv_ref[...] = dinv
        g_ref[...] = h_ref[...] * dinv

    return pl.pallas_call(
        body,
        out_shape=[
            jax.ShapeDtypeStruct((N, D_HID), jnp.float32),
            jax.ShapeDtypeStruct((N, 1), jnp.float32),
        ],
    )(h1, deg_parts)


def _tc_layer2_in(parts1, g1, dinv, b1r, w2p):
    def body(p_ref, g1_ref, dinv_ref, b1_ref, w2_ref, g2_ref):
        s1 = p_ref[0, :N] + p_ref[1, :N] + g1_ref[...]
        a1 = jnp.maximum(s1 * dinv_ref[...] + b1_ref[...], 0.0)
        h2 = jnp.dot(a1, w2_ref[...], preferred_element_type=jnp.float32)
        g2_ref[...] = h2 * dinv_ref[...]

    return pl.pallas_call(
        body, out_shape=jax.ShapeDtypeStruct((N, D_OUT2), jnp.float32)
    )(parts1, g1, dinv, b1r, w2p)


def _tc_final(parts2, g2, dinv, b2r):
    def body(p_ref, g2_ref, dinv_ref, b2_ref, o_ref):
        res = ((p_ref[0, :N] + p_ref[1, :N] + g2_ref[...])
               * dinv_ref[...] + b2_ref[...])
        o_ref[...] = res[:, :2]

    return pl.pallas_call(
        body, out_shape=jax.ShapeDtypeStruct((N, 2), jnp.float32)
    )(parts2, g2, dinv, b2r)


def kernel(x, edge_index, W1, b1, W2, b2):
    pad = E_PAD - E
    src2 = jnp.concatenate(
        [edge_index[0], jnp.zeros((pad,), jnp.int32)]).reshape(NCHT, CH)
    dst2 = jnp.concatenate(
        [edge_index[1], jnp.full((pad,), N, jnp.int32)]).reshape(NCHT, CH)

    ones_h = jnp.ones((CH, 1), jnp.float32)
    zeros_deg = jnp.zeros((ROWS_PT, 1), jnp.float32)
    zeros64 = jnp.zeros((CH, D_HID), jnp.float32)
    zeros8 = jnp.zeros((CH, D_OUT2), jnp.float32)
    b1r = b1.reshape(1, D_HID)
    b2r = jnp.pad(b2, (0, D_OUT2 - b2.shape[0])).reshape(1, D_OUT2)
    w2p = jnp.pad(W2, ((0, 0), (0, D_OUT2 - W2.shape[1])))

    deg_parts = _sc_degree(dst2, ones_h, zeros_deg)
    h1 = _tc_matmul1(x, W1)

    g1, dinv = _tc_scale(h1, deg_parts)

    parts1 = _sc_aggregate(g1, src2, dst2, zeros64, D_HID, SPLIT1)
    g2 = _tc_layer2_in(parts1, g1, dinv, b1r, w2p)

    parts2 = _sc_aggregate(g2, src2, dst2, zeros8, D_OUT2, SPLIT2)
    return _tc_final(parts2, g2, dinv, b2r)
